# baseline (speedup 1.0000x reference)
import functools

import jax
import jax.numpy as jnp
from jax.experimental import pallas as pl
from jax.experimental.pallas import tpu as pltpu

_SCALING = 32.0 / 16.0


def _ffn_body(E, R, F, FB,
              logits_ref, x_ref, w1_ref, w3_ref, w2_ref,
              a1_ref, a3_ref, a2_ref, b1t_ref, b3t_ref, b2t_ref,
              out_ref,
              xT_ref, maskT_ref, la1T_ref, la3T_ref, la2T_ref, oT_ref):
    f32 = jnp.float32
    bf16 = jnp.bfloat16
    ER = E * R
    i = pl.program_id(0)
    nstep = F // FB

    @pl.when(i == 0)
    def _prologue():
        xT = jnp.transpose(x_ref[...].astype(bf16))
        xT_ref[...] = xT
        logits = logits_ref[...]
        m = jnp.max(logits, axis=-1, keepdims=True)
        ids_e = jax.lax.broadcasted_iota(jnp.int32, logits.shape, 1)
        e = jnp.min(jnp.where(logits == m, ids_e, E), axis=-1,
                    keepdims=True)
        ids = jax.lax.broadcasted_iota(jnp.int32, (logits.shape[0], ER), 1)
        mask = (ids // R == e).astype(f32)
        maskT = jnp.transpose(mask)
        maskT_ref[...] = maskT
        a1b = a1_ref[...].astype(bf16)
        la1T_ref[...] = (jnp.dot(a1b, xT, preferred_element_type=f32)
                         * maskT).astype(bf16)
        a3b = a3_ref[...].astype(bf16)
        la3T_ref[...] = (jnp.dot(a3b, xT, preferred_element_type=f32)
                         * maskT).astype(bf16)

    xT = xT_ref[...]
    w1b = w1_ref[...].astype(bf16)
    gT = (jnp.dot(w1b, xT, preferred_element_type=f32)
          + jnp.dot(b1t_ref[...], la1T_ref[...], preferred_element_type=f32))
    w3b = w3_ref[...].astype(bf16)
    uT = (jnp.dot(w3b, xT, preferred_element_type=f32)
          + jnp.dot(b3t_ref[...], la3T_ref[...], preferred_element_type=f32))
    hT = ((gT * jax.lax.logistic(gT)) * uT).astype(bf16)

    w2b = w2_ref[...].astype(bf16)
    o_contrib = jnp.dot(w2b, hT, preferred_element_type=f32)
    a2b = a2_ref[...].astype(bf16)
    la2_contrib = jnp.dot(a2b, hT, preferred_element_type=f32)

    @pl.when(i == 0)
    def _init_acc():
        oT_ref[...] = o_contrib
        la2T_ref[...] = la2_contrib

    @pl.when(i > 0)
    def _acc():
        oT_ref[...] += o_contrib
        la2T_ref[...] += la2_contrib

    @pl.when(i == nstep - 1)
    def _epilogue():
        la2m = (la2T_ref[...] * maskT_ref[...]).astype(bf16)
        oT = oT_ref[...] + jnp.dot(b2t_ref[...], la2m,
                                   preferred_element_type=f32)
        out_ref[...] = jnp.transpose(oT)


def kernel(data, gate_weight, w1, w2, w3,
           lora_a1, lora_b1, lora_a3, lora_b3, lora_a2, lora_b2):
    T, D = data.shape
    F = w1.shape[0]
    E, R, _ = lora_a1.shape
    ER = E * R
    s = _SCALING
    f32 = jnp.float32
    bf16 = jnp.bfloat16

    router_logits = data @ gate_weight.T

    a1r = lora_a1.reshape(ER, D)
    a3r = lora_a3.reshape(ER, D)
    a2r = lora_a2.reshape(ER, F)
    b1t = (lora_b1.transpose(1, 0, 2).reshape(F, ER) * s).astype(bf16)
    b3t = (lora_b3.transpose(1, 0, 2).reshape(F, ER) * s).astype(bf16)
    b2t = (lora_b2.transpose(1, 0, 2).reshape(D, ER) * s).astype(bf16)

    FB = 128
    grid = (F // FB,)
    rep = lambda i: (0, 0)
    frow = lambda i: (i, 0)
    fcol = lambda i: (0, i)

    out = pl.pallas_call(
        functools.partial(_ffn_body, E, R, F, FB),
        grid=grid,
        in_specs=[
            pl.BlockSpec((T, E), rep),
            pl.BlockSpec((T, D), rep),
            pl.BlockSpec((FB, D), frow),
            pl.BlockSpec((FB, D), frow),
            pl.BlockSpec((D, FB), fcol),
            pl.BlockSpec((ER, D), rep),
            pl.BlockSpec((ER, D), rep),
            pl.BlockSpec((ER, FB), fcol),
            pl.BlockSpec((FB, ER), frow),
            pl.BlockSpec((FB, ER), frow),
            pl.BlockSpec((D, ER), rep),
        ],
        out_specs=pl.BlockSpec((T, D), rep),
        out_shape=jax.ShapeDtypeStruct((T, D), data.dtype),
        scratch_shapes=[
            pltpu.VMEM((D, T), bf16),
            pltpu.VMEM((ER, T), f32),
            pltpu.VMEM((ER, T), bf16),
            pltpu.VMEM((ER, T), bf16),
            pltpu.VMEM((ER, T), f32),
            pltpu.VMEM((D, T), f32),
        ],
    )(router_logits, data, w1, w3, w2, a1r, a3r, a2r, b1t, b3t, b2t)
    return out, router_logits


# speedup vs baseline: 1.4347x; 1.4347x over previous
import functools

import jax
import jax.numpy as jnp
from jax.experimental import pallas as pl
from jax.experimental.pallas import tpu as pltpu

_SCALING = 32.0 / 16.0


def _ffn_body(E, R, F, FB, T,
              logits_ref, x_ref, w1_ref, w3_ref, w2_ref,
              a1_ref, a3_ref, a2_ref, b1t_ref, b3t_ref, b2t_ref,
              out_ref,
              xT_ref, maskT_ref, la1T_ref, la3T_ref, hT_ref, w2s_ref):
    f32 = jnp.float32
    bf16 = jnp.bfloat16
    ER = E * R
    i = pl.program_id(0)
    nstep = F // FB

    @pl.when(i == 0)
    def _prologue():
        xT = jnp.transpose(x_ref[...].astype(bf16))
        xT_ref[...] = xT
        logits = logits_ref[...]
        m = jnp.max(logits, axis=-1, keepdims=True)
        ids_e = jax.lax.broadcasted_iota(jnp.int32, logits.shape, 1)
        e = jnp.min(jnp.where(logits == m, ids_e, E), axis=-1,
                    keepdims=True)
        ids = jax.lax.broadcasted_iota(jnp.int32, (logits.shape[0], ER), 1)
        mask = (ids // R == e).astype(bf16)
        maskT = jnp.transpose(mask)
        maskT_ref[...] = maskT
        a1b = a1_ref[...].astype(bf16)
        la1T_ref[...] = (jnp.dot(a1b, xT, preferred_element_type=f32)
                         * maskT.astype(f32)).astype(bf16)
        a3b = a3_ref[...].astype(bf16)
        la3T_ref[...] = (jnp.dot(a3b, xT, preferred_element_type=f32)
                         * maskT.astype(f32)).astype(bf16)

    xT = xT_ref[...]
    w1b = w1_ref[...].astype(bf16)
    gT = (jnp.dot(w1b, xT, preferred_element_type=f32)
          + jnp.dot(b1t_ref[...], la1T_ref[...], preferred_element_type=f32))
    w3b = w3_ref[...].astype(bf16)
    uT = (jnp.dot(w3b, xT, preferred_element_type=f32)
          + jnp.dot(b3t_ref[...], la3T_ref[...], preferred_element_type=f32))
    hT_ref[pl.ds(i * FB, FB), :] = \
        ((gT * jax.lax.logistic(gT)) * uT).astype(bf16)
    w2s_ref[i] = w2_ref[...].astype(bf16)

    @pl.when(i == nstep - 1)
    def _epilogue():
        a2b = a2_ref[...].astype(bf16)
        TH = T // 2
        for h in range(2):
            sl = slice(h * TH, (h + 1) * TH)
            hT_h = hT_ref[:, sl]
            la2 = (jnp.dot(a2b, hT_h, preferred_element_type=f32)
                   * maskT_ref[:, sl].astype(f32)).astype(bf16)
            oT_h = jnp.dot(b2t_ref[...], la2, preferred_element_type=f32)
            for j in range(nstep):
                oT_h += jnp.dot(w2s_ref[j], hT_ref[pl.ds(j * FB, FB), sl],
                                preferred_element_type=f32)
            out_ref[sl, :] = jnp.transpose(oT_h)


def kernel(data, gate_weight, w1, w2, w3,
           lora_a1, lora_b1, lora_a3, lora_b3, lora_a2, lora_b2):
    T, D = data.shape
    F = w1.shape[0]
    E, R, _ = lora_a1.shape
    ER = E * R
    s = _SCALING
    f32 = jnp.float32
    bf16 = jnp.bfloat16

    router_logits = data @ gate_weight.T

    a1r = lora_a1.reshape(ER, D)
    a3r = lora_a3.reshape(ER, D)
    a2r = lora_a2.reshape(ER, F)
    b1t = (lora_b1.transpose(1, 0, 2).reshape(F, ER) * s).astype(bf16)
    b3t = (lora_b3.transpose(1, 0, 2).reshape(F, ER) * s).astype(bf16)
    b2t = (lora_b2.transpose(1, 0, 2).reshape(D, ER) * s).astype(bf16)

    FB = 256
    grid = (F // FB,)
    rep = lambda i: (0, 0)
    frow = lambda i: (i, 0)
    fcol = lambda i: (0, i)

    out = pl.pallas_call(
        functools.partial(_ffn_body, E, R, F, FB, T),
        grid=grid,
        in_specs=[
            pl.BlockSpec((T, E), rep),
            pl.BlockSpec((T, D), rep),
            pl.BlockSpec((FB, D), frow),
            pl.BlockSpec((FB, D), frow),
            pl.BlockSpec((D, FB), fcol),
            pl.BlockSpec((ER, D), rep),
            pl.BlockSpec((ER, D), rep),
            pl.BlockSpec((ER, F), rep),
            pl.BlockSpec((FB, ER), frow),
            pl.BlockSpec((FB, ER), frow),
            pl.BlockSpec((D, ER), rep),
        ],
        out_specs=pl.BlockSpec((T, D), rep),
        out_shape=jax.ShapeDtypeStruct((T, D), data.dtype),
        scratch_shapes=[
            pltpu.VMEM((D, T), bf16),
            pltpu.VMEM((ER, T), bf16),
            pltpu.VMEM((ER, T), bf16),
            pltpu.VMEM((ER, T), bf16),
            pltpu.VMEM((F, T), bf16),
            pltpu.VMEM((F // FB, D, FB), bf16),
        ],
    )(router_logits, data, w1, w3, w2, a1r, a3r, a2r, b1t, b3t, b2t)
    return out, router_logits
